# baseline (speedup 1.0000x reference)
import jax
import jax.numpy as jnp
from jax import lax
from jax.experimental import pallas as pl
from jax.experimental.pallas import tpu as pltpu
from jax.experimental.pallas import tpu_sc as plsc

B, T, D = 4096, 200, 128
NC, NS = 2, 16
NW = NC * NS
BPW = B // NW
LANES = 16
NG = BPW // LANES
NCH = 5
TCH = T // NCH
HALF = BPW // 2


def _sc_body(x_hbm, maskT_hbm, out_hbm,
             mb0, mb1, idx_a, idx_b, rows_a, rows_b,
             sm0, sm1, sg, so):
    wid = lax.axis_index("s") * NC + lax.axis_index("c")
    base = wid * BPW
    bufs = (mb0, mb1)
    sems = (sm0, sm1)

    def start(c):
        return pltpu.async_copy(
            maskT_hbm.at[0, pl.ds(c * TCH, TCH), pl.ds(base, BPW)],
            bufs[c % 2], sems[c % 2])

    cps = [start(0), start(1)]
    lane = lax.iota(jnp.int32, LANES)
    accs = (jnp.zeros((LANES,), jnp.int32),) * NG

    for c in range(NCH):
        cps[c % 2].wait()
        buf = bufs[c % 2]

        def t_body(t, a):
            return tuple(
                acc + buf[t, pl.ds(g * LANES, LANES)]
                for g, acc in enumerate(a)
            )

        accs = lax.fori_loop(0, TCH, t_body, accs)
        if c + 2 < NCH:
            cps[c % 2] = start(c + 2)

    for g in range(NG):
        s = accs[g]
        row = jnp.where(s == 0, T - 1, s - 1)
        flat = (base + g * LANES + lane) * T + row
        if g < NG // 2:
            idx_a[pl.ds(g * LANES, LANES)] = flat
        else:
            idx_b[pl.ds((g - NG // 2) * LANES, LANES)] = flat

    pltpu.async_copy(x_hbm.at[idx_a], rows_a, sg).wait()
    co_a = pltpu.async_copy(rows_a, out_hbm.at[pl.ds(base, HALF)], so)
    pltpu.async_copy(x_hbm.at[idx_b], rows_b, sg).wait()
    co_b = pltpu.async_copy(rows_b, out_hbm.at[pl.ds(base + HALF, HALF)], so)
    co_a.wait()
    co_b.wait()


def kernel(x, dim, mask):
    del dim
    maskT = jnp.transpose(mask, (1, 2, 0))
    mesh = plsc.VectorSubcoreMesh(core_axis_name="c", subcore_axis_name="s")
    run = pl.kernel(
        _sc_body,
        out_type=jax.ShapeDtypeStruct((B, D), jnp.float32),
        mesh=mesh,
        scratch_types=[
            pltpu.VMEM((TCH, BPW), jnp.int32),
            pltpu.VMEM((TCH, BPW), jnp.int32),
            pltpu.VMEM((HALF,), jnp.int32),
            pltpu.VMEM((HALF,), jnp.int32),
            pltpu.VMEM((HALF, D), jnp.float32),
            pltpu.VMEM((HALF, D), jnp.float32),
            pltpu.SemaphoreType.DMA,
            pltpu.SemaphoreType.DMA,
            pltpu.SemaphoreType.DMA,
            pltpu.SemaphoreType.DMA,
        ],
    )
    out = run(x.reshape(B * T, D), maskT)
    return out.reshape(B, 1, D)


# speedup vs baseline: 1.0249x; 1.0249x over previous
import jax
import jax.numpy as jnp
from jax import lax
from jax.experimental import pallas as pl
from jax.experimental.pallas import tpu as pltpu
from jax.experimental.pallas import tpu_sc as plsc

B, T, D = 4096, 200, 128
NC, NS = 2, 16
NW = NC * NS
BPW = B // NW
LANES = 16
NG = BPW // LANES
NCH = 2
TCHS = (96, 104)
TOFF = (0, 96)
HALF = BPW // 2


def _sc_body(x_hbm, maskT_hbm, out_hbm,
             mb0, mb1, idx_a, idx_b, rows_a, rows_b,
             sm0, sm1, sg, so):
    wid = lax.axis_index("s") * NC + lax.axis_index("c")
    base = wid * BPW
    bufs = (mb0, mb1)
    sems = (sm0, sm1)

    def start(c):
        return pltpu.async_copy(
            maskT_hbm.at[0, pl.ds(TOFF[c], TCHS[c]), pl.ds(base, BPW)],
            bufs[c], sems[c])

    cps = [start(0), start(1)]
    lane = lax.iota(jnp.int32, LANES)
    accs = (jnp.zeros((LANES,), jnp.int32),) * NG

    for c in range(NCH):
        cps[c].wait()
        buf = bufs[c]

        def t_body(t, a):
            return tuple(
                acc + buf[t, pl.ds(g * LANES, LANES)]
                for g, acc in enumerate(a)
            )

        accs = lax.fori_loop(0, TCHS[c], t_body, accs)

    for g in range(NG):
        s = accs[g]
        row = jnp.where(s == 0, T - 1, s - 1)
        flat = (base + g * LANES + lane) * T + row
        if g < NG // 2:
            idx_a[pl.ds(g * LANES, LANES)] = flat
        else:
            idx_b[pl.ds((g - NG // 2) * LANES, LANES)] = flat

    pltpu.async_copy(x_hbm.at[idx_a], rows_a, sg).wait()
    co_a = pltpu.async_copy(rows_a, out_hbm.at[pl.ds(base, HALF)], so)
    pltpu.async_copy(x_hbm.at[idx_b], rows_b, sg).wait()
    co_b = pltpu.async_copy(rows_b, out_hbm.at[pl.ds(base + HALF, HALF)], so)
    co_a.wait()
    co_b.wait()


def kernel(x, dim, mask):
    del dim
    maskT = jnp.transpose(mask, (1, 2, 0))
    mesh = plsc.VectorSubcoreMesh(core_axis_name="c", subcore_axis_name="s")
    run = pl.kernel(
        _sc_body,
        out_type=jax.ShapeDtypeStruct((B, D), jnp.float32),
        mesh=mesh,
        scratch_types=[
            pltpu.VMEM((TCHS[0], BPW), jnp.int32),
            pltpu.VMEM((TCHS[1], BPW), jnp.int32),
            pltpu.VMEM((HALF,), jnp.int32),
            pltpu.VMEM((HALF,), jnp.int32),
            pltpu.VMEM((HALF, D), jnp.float32),
            pltpu.VMEM((HALF, D), jnp.float32),
            pltpu.SemaphoreType.DMA,
            pltpu.SemaphoreType.DMA,
            pltpu.SemaphoreType.DMA,
            pltpu.SemaphoreType.DMA,
        ],
    )
    out = run(x.reshape(B * T, D), maskT)
    return out.reshape(B, 1, D)
